# Initial kernel scaffold; baseline (speedup 1.0000x reference)
import jax
import jax.numpy as jnp
from jax.experimental import pallas as pl


def kernel(x, edge_index, edge_weight, params):
    raise NotImplementedError("write your pallas kernel here")


# speedup vs baseline: 1.1571x; 1.1571x over previous
import functools

import jax
import jax.numpy as jnp
from jax import lax
from jax.experimental import pallas as pl
from jax.experimental.pallas import tpu as pltpu
from jax.experimental.pallas import tpu_sc as plsc

N = 50000
E = 800000
NPAD = 50048
B = 1024
NBATCH = 50
EPAD = 16 * B * NBATCH
ROWS2D = EPAD // 128
EPS = 1e-5
_f32 = jnp.float32
_i32 = jnp.int32


@functools.cache
def _make_agg(cmul: int, nchunks: int):
    cpc = max(nchunks // 2, 1)
    mesh = plsc.VectorSubcoreMesh(core_axis_name="c", subcore_axis_name="s")
    scratch = [
        pltpu.VMEM((8, 128), _i32),
        pltpu.VMEM((8, 128), _i32),
        pltpu.VMEM((B, 16), _f32),
        pltpu.VMEM((8, 128), _i32),
        pltpu.VMEM((B, 16), _f32),
        pltpu.VMEM((184, 16), _f32),
        pltpu.VMEM_SHARED((NPAD, 16), _f32),
    ]
    out_type = [jax.ShapeDtypeStruct((NPAD, 16), _f32) for _ in range(nchunks)]

    def body(table, row2d, col2d, wexp, zeros, *rest):
        outs = rest[:nchunks]
        row_v, col_v, wex_v, gidx_v, rows_v, tbuf, acc = rest[nchunks:]
        cid = lax.axis_index("c")
        sid = lax.axis_index("s")

        def chunk_body(k, out_ref):
            @pl.when(sid == 0)
            def _zero():
                pltpu.sync_copy(zeros, acc)
            plsc.subcore_barrier()

            def batch(bi, _):
                r0 = sid * (NBATCH * 8) + bi * 8
                e0 = sid * (NBATCH * B) + bi * B
                pltpu.sync_copy(row2d.at[pl.ds(r0, 8)], row_v)
                pltpu.sync_copy(col2d.at[pl.ds(r0, 8)], col_v)
                pltpu.sync_copy(wexp.at[pl.ds(e0, B)], wex_v)
                if cmul != 1:
                    def gix(j, _):
                        def gslice(q, _):
                            v = row_v[j, pl.ds(q * 16, 16)] * cmul + k
                            gidx_v[j, pl.ds(q * 16, 16)] = v
                            return 0
                        return lax.fori_loop(0, 8, gslice, 0, unroll=8)
                    lax.fori_loop(0, 8, gix, 0)

                def gath(j, _):
                    idxr = gidx_v.at[j] if cmul != 1 else row_v.at[j]
                    pltpu.sync_copy(table.at[idxr],
                                    rows_v.at[pl.ds(j * 128, 128)])
                    return 0
                lax.fori_loop(0, 8, gath, 0)

                def scale_one(e, _):
                    rows_v[e, pl.ds(0, 16)] = (rows_v[e, pl.ds(0, 16)]
                                               * wex_v[e, pl.ds(0, 16)])
                    return 0
                lax.fori_loop(0, B, scale_one, 0, unroll=8)

                def scat(j, _):
                    pltpu.sync_copy(rows_v.at[pl.ds(j * 128, 128)],
                                    acc.at[col_v.at[j]], add=True)
                    return 0
                lax.fori_loop(0, 8, scat, 0)
                return 0

            lax.fori_loop(0, NBATCH, batch, 0)
            plsc.subcore_barrier()

            def wo(tq, _):
                off = sid * 3128 + tq * 184
                pltpu.sync_copy(acc.at[pl.ds(off, 184)], tbuf)
                pltpu.sync_copy(tbuf, out_ref.at[pl.ds(off, 184)])
                return 0
            lax.fori_loop(0, 17, wo, 0)
            plsc.subcore_barrier()

        for k in range(nchunks):
            owner = k // cpc

            @pl.when(cid == owner)
            def _proc(k=k, out_ref=outs[k]):
                chunk_body(k, out_ref)

    return pl.kernel(body, out_type=out_type, mesh=mesh,
                     scratch_types=scratch,
                     compiler_params=pltpu.CompilerParams(
                         use_tc_tiling_on_sc=False))


@functools.cache
def _make_edge_gather():
    mesh = plsc.VectorSubcoreMesh(core_axis_name="c", subcore_axis_name="s")
    scratch = [
        pltpu.VMEM((8, 128), _i32),
        pltpu.VMEM((1024, 16), _f32),
    ]
    out_type = [jax.ShapeDtypeStruct((EPAD, 16), _f32) for _ in range(2)]
    per_w = EPAD // 32
    nb = per_w // 1024

    def body(h12, row2d, col2d, er, ec, idx_v, buf):
        cid = lax.axis_index("c")
        sid = lax.axis_index("s")
        wid = sid * 2 + cid

        def batch(bi, _):
            r0 = wid * (per_w // 128) + bi * 8
            eoff = wid * per_w + bi * 1024
            for src2d, dst in ((row2d, er), (col2d, ec)):
                pltpu.sync_copy(src2d.at[pl.ds(r0, 8)], idx_v)

                def g1(j, _):
                    pltpu.sync_copy(h12.at[idx_v.at[j]],
                                    buf.at[pl.ds(j * 128, 128)])
                    return 0
                lax.fori_loop(0, 8, g1, 0)
                pltpu.sync_copy(buf, dst.at[pl.ds(eoff, 1024)])
            return 0
        lax.fori_loop(0, nb, batch, 0)

    return pl.kernel(body, out_type=out_type, mesh=mesh,
                     scratch_types=scratch,
                     compiler_params=pltpu.CompilerParams(
                         use_tc_tiling_on_sc=False))


def _tc_stats(arrs, br):
    r = arrs[0].shape[0]
    ktot = sum(a.shape[1] for a in arrs)
    grid = r // br

    def kern(*refs):
        ins, out = refs[:-1], refs[-1]
        x = jnp.concatenate([rf[...] for rf in ins], axis=1)
        blk = jnp.stack([jnp.sum(x, 0), jnp.sum(x * x, 0)])

        @pl.when(pl.program_id(0) == 0)
        def _():
            out[...] = jnp.zeros_like(out)
        out[...] += blk

    return pl.pallas_call(
        kern, grid=(grid,),
        in_specs=[pl.BlockSpec((br, a.shape[1]), lambda i: (i, 0))
                  for a in arrs],
        out_specs=pl.BlockSpec((2, ktot), lambda i: (0, 0)),
        out_shape=jax.ShapeDtypeStruct((2, ktot), _f32))(*arrs)


def _tc_mm(arrs, s, t, w, relu, br=544):
    r = arrs[0].shape[0]
    ktot = sum(a.shape[1] for a in arrs)
    fout = w.shape[1]
    grid = r // br
    have_aff = s is not None
    extra = [s, t] if have_aff else []

    def kern(*refs):
        ins = refs[:len(arrs)]
        pos = len(arrs)
        if have_aff:
            s_ref, t_ref = refs[pos], refs[pos + 1]
            pos += 2
        w_ref, out = refs[pos], refs[pos + 1]
        x = jnp.concatenate([rf[...] for rf in ins], axis=1)
        if have_aff:
            x = x * s_ref[...] + t_ref[...]
        if relu:
            x = jnp.maximum(x, 0.0)
        out[...] = jnp.dot(x, w_ref[...], preferred_element_type=_f32)

    in_specs = [pl.BlockSpec((br, a.shape[1]), lambda i: (i, 0))
                for a in arrs]
    if have_aff:
        in_specs += [pl.BlockSpec((1, ktot), lambda i: (0, 0))] * 2
    in_specs += [pl.BlockSpec((ktot, fout), lambda i: (0, 0))]
    return pl.pallas_call(
        kern, grid=(grid,),
        in_specs=in_specs,
        out_specs=pl.BlockSpec((br, fout), lambda i: (i, 0)),
        out_shape=jax.ShapeDtypeStruct((r, fout), _f32))(*arrs, *extra, w)


def _tc_affine16(a, s, t, br=544):
    r = a.shape[0]
    grid = r // br

    def kern(a_ref, s_ref, t_ref, out):
        out[...] = a_ref[...] * s_ref[...] + t_ref[...]

    return pl.pallas_call(
        kern, grid=(grid,),
        in_specs=[pl.BlockSpec((br, 16), lambda i: (i, 0)),
                  pl.BlockSpec((1, 16), lambda i: (0, 0)),
                  pl.BlockSpec((1, 16), lambda i: (0, 0))],
        out_specs=pl.BlockSpec((br, 16), lambda i: (i, 0)),
        out_shape=jax.ShapeDtypeStruct((r, 16), _f32))(a, s, t)


def _tc_mlp_stats(er, ec, wa, wb, be=1000):
    grid = E // be

    def kern(er_ref, ec_ref, wa_ref, wb_ref, out):
        e_r, e_c = er_ref[...], ec_ref[...]
        wa_, wb_ = wa_ref[...], wb_ref[...]
        u1 = (jnp.dot(e_r, wa_, preferred_element_type=_f32)
              + jnp.dot(e_c, wb_, preferred_element_type=_f32))
        u2 = (jnp.dot(e_c, wa_, preferred_element_type=_f32)
              + jnp.dot(e_r, wb_, preferred_element_type=_f32))
        blk = jnp.stack([jnp.sum(u1, 0), jnp.sum(u1 * u1, 0),
                         jnp.sum(u2, 0), jnp.sum(u2 * u2, 0)])

        @pl.when(pl.program_id(0) == 0)
        def _():
            out[...] = jnp.zeros_like(out)
        out[...] += blk

    return pl.pallas_call(
        kern, grid=(grid,),
        in_specs=[pl.BlockSpec((be, 16), lambda i: (i, 0)),
                  pl.BlockSpec((be, 16), lambda i: (i, 0)),
                  pl.BlockSpec((16, 128), lambda i: (0, 0)),
                  pl.BlockSpec((16, 128), lambda i: (0, 0))],
        out_specs=pl.BlockSpec((4, 128), lambda i: (0, 0)),
        out_shape=jax.ShapeDtypeStruct((4, 128), _f32))(er, ec, wa, wb)


def _tc_mlp_final(er, ec, wa, wb, s1, t1, s2, t2, w2, b2, be=8000):
    grid = E // be

    def kern(er_ref, ec_ref, wa_ref, wb_ref, s1r, t1r, s2r, t2r, w2r, b2r,
             out):
        e_r, e_c = er_ref[...], ec_ref[...]
        wa_, wb_ = wa_ref[...], wb_ref[...]
        u1 = (jnp.dot(e_r, wa_, preferred_element_type=_f32)
              + jnp.dot(e_c, wb_, preferred_element_type=_f32))
        u2 = (jnp.dot(e_c, wa_, preferred_element_type=_f32)
              + jnp.dot(e_r, wb_, preferred_element_type=_f32))
        h1 = jnp.maximum(u1 * s1r[...] + t1r[...], 0.0)
        h2 = jnp.maximum(u2 * s2r[...] + t2r[...], 0.0)
        p1 = jnp.sum(h1 * w2r[...], axis=1)
        p2 = jnp.sum(h2 * w2r[...], axis=1)
        z = 0.5 * (p1 + p2) + b2r[0, 0]
        out[...] = (1.0 / (1.0 + jnp.exp(-z))).reshape(out.shape)

    cst = lambda shape: pl.BlockSpec(shape, lambda i: (0, 0))
    return pl.pallas_call(
        kern, grid=(grid,),
        in_specs=[pl.BlockSpec((be, 16), lambda i: (i, 0)),
                  pl.BlockSpec((be, 16), lambda i: (i, 0)),
                  cst((16, 128)), cst((16, 128)),
                  cst((1, 128)), cst((1, 128)), cst((1, 128)), cst((1, 128)),
                  cst((1, 128)), cst((1, 1))],
        out_specs=pl.BlockSpec((be // 1000, 1000), lambda i: (i, 0)),
        out_shape=jax.ShapeDtypeStruct((E // 1000, 1000), _f32))(
            er, ec, wa, wb, s1, t1, s2, t2, w2, b2)


def kernel(x, edge_index, edge_weight, params):
    row = edge_index[0].astype(_i32)
    col = edge_index[1].astype(_i32)
    w = edge_weight.astype(_f32)
    pad = EPAD - E
    row2d = jnp.concatenate([row, jnp.zeros((pad,), _i32)]).reshape(ROWS2D, 128)
    col2d = jnp.concatenate([col, jnp.zeros((pad,), _i32)]).reshape(ROWS2D, 128)
    w1d = jnp.concatenate([w, jnp.zeros((pad,), _f32)])
    wexp = jnp.broadcast_to(w1d[:, None], (EPAD, 16))
    zeros16 = jnp.zeros((NPAD, 16), _f32)
    convs = params['convs']
    agg1 = _make_agg(1, 1)
    agg8 = _make_agg(8, 8)

    st_x = _tc_stats([x], 1000)
    m0 = st_x[0] / N
    v0 = st_x[1] / N - m0 * m0
    s0 = params['bn0_g'] * lax.rsqrt(v0 + EPS)
    t0 = params['bn0_b'] - m0 * s0
    w0 = convs[0]['W']
    weff = (jnp.zeros((16, 128), _f32)
            .at[0].set(s0[0] * w0[0])
            .at[1].set(s0[1] * w0[1])
            .at[2].set(t0[0] * w0[0] + t0[1] * w0[1]))

    tbl0 = (jnp.zeros((NPAD, 16), _f32)
            .at[:N, 0:2].set(x)
            .at[:N, 2].set(1.0))
    g = agg1(tbl0, row2d, col2d, wexp, zeros16)[0]
    cur = [_tc_mm([g], None, None, weff, False)]

    a11 = None
    for i in range(1, 12):
        st = _tc_stats(cur, 544)
        m = st[0] / N
        v = st[1] / N - m * m
        s = convs[i - 1]['bn_g'] * lax.rsqrt(v + EPS)
        t = convs[i - 1]['bn_b'] - m * s
        if i < 11:
            hw = _tc_mm(cur, s[None], t[None], convs[i]['W'], True)
            cur = list(agg8(hw.reshape(NPAD * 8, 16), row2d, col2d, wexp,
                            zeros16))
        else:
            hw11 = _tc_mm(cur, s[None], t[None], convs[11]['W'], True)
            a11 = agg1(hw11, row2d, col2d, wexp, zeros16)[0]

    st11 = _tc_stats([a11], 544)
    m = st11[0] / N
    v = st11[1] / N - m * m
    s = convs[11]['bn_g'] * lax.rsqrt(v + EPS)
    t = convs[11]['bn_b'] - m * s
    h12 = _tc_affine16(a11, s[None], t[None])

    e_r, e_c = _make_edge_gather()(h12, row2d, col2d)

    w1 = params['mlp_W1']
    w1a, w1b = w1[:16], w1[16:]
    stm = _tc_mlp_stats(e_r, e_c, w1a, w1b)
    m1 = stm[0] / E
    v1 = stm[1] / E - m1 * m1
    m2 = stm[2] / E
    v2 = stm[3] / E - m2 * m2
    gm, bm = params['mlp_bn_g'], params['mlp_bn_b']
    s1 = gm * lax.rsqrt(v1 + EPS)
    t1 = bm - m1 * s1
    s2 = gm * lax.rsqrt(v2 + EPS)
    t2 = bm - m2 * s2
    w2row = params['mlp_W2'][:, 0][None]
    b2v = params['mlp_b2'].reshape(1, 1)
    out2 = _tc_mlp_final(e_r, e_c, w1a, w1b, s1[None], t1[None], s2[None],
                         t2[None], w2row, b2v)
    return out2.reshape(E)


# speedup vs baseline: 1.5933x; 1.3770x over previous
import functools

import jax
import jax.numpy as jnp
from jax import lax
from jax.experimental import pallas as pl
from jax.experimental.pallas import tpu as pltpu
from jax.experimental.pallas import tpu_sc as plsc

N = 50000
E = 800000
NPAD = 50048
B = 2048
NBATCH = 25
EPAD = 16 * B * NBATCH
ROWS2D = EPAD // 128
EPS = 1e-5
_f32 = jnp.float32
_i32 = jnp.int32


@functools.cache
def _make_agg(cmul: int, nchunks: int):
    cpc = max(nchunks // 2, 1)
    mesh = plsc.VectorSubcoreMesh(core_axis_name="c", subcore_axis_name="s")
    scratch = [
        pltpu.VMEM((16, 128), _i32),
        pltpu.VMEM((16, 128), _i32),
        pltpu.VMEM((B, 16), _f32),
        pltpu.VMEM((16, 128), _i32),
        pltpu.VMEM((B, 16), _f32),
        pltpu.VMEM((184, 16), _f32),
        pltpu.VMEM_SHARED((NPAD, 16), _f32),
        pltpu.SemaphoreType.DMA,
        pltpu.SemaphoreType.DMA,
        pltpu.SemaphoreType.DMA,
    ]
    out_type = [jax.ShapeDtypeStruct((NPAD, 16), _f32) for _ in range(nchunks)]

    def body(table, row2d, col2d, wexp, zeros, *rest):
        outs = rest[:nchunks]
        (row_v, col_v, wex_v, gidx_v, rows_v, tbuf, acc,
         sem_i, sem_g, sem_s) = rest[nchunks:]
        cid = lax.axis_index("c")
        sid = lax.axis_index("s")

        def chunk_body(k, out_ref):
            @pl.when(sid == 0)
            def _zero():
                pltpu.sync_copy(zeros, acc)
            plsc.subcore_barrier()

            def drain_scat():
                for j in range(16):
                    pltpu.make_async_copy(
                        rows_v.at[pl.ds(j * 128, 128)],
                        acc.at[col_v.at[j]], sem_s).wait()

            def batch(bi, _):
                r0 = sid * (NBATCH * 16) + bi * 16
                e0 = sid * (NBATCH * B) + bi * B

                @pl.when(bi > 0)
                def _drain():
                    drain_scat()

                d1 = pltpu.async_copy(row2d.at[pl.ds(r0, 16)], row_v, sem_i)
                d2 = pltpu.async_copy(col2d.at[pl.ds(r0, 16)], col_v, sem_i)
                d3 = pltpu.async_copy(wexp.at[pl.ds(e0, B)], wex_v, sem_i)
                d1.wait()
                d2.wait()
                d3.wait()
                if cmul != 1:
                    for j in range(16):
                        def gslice(q, _, j=j):
                            v = row_v[j, pl.ds(q * 16, 16)] * cmul + k
                            gidx_v[j, pl.ds(q * 16, 16)] = v
                            return 0
                        lax.fori_loop(0, 8, gslice, 0, unroll=8)

                gds = []
                for j in range(16):
                    idxr = gidx_v.at[j] if cmul != 1 else row_v.at[j]
                    gds.append(pltpu.async_copy(
                        table.at[idxr], rows_v.at[pl.ds(j * 128, 128)],
                        sem_g))
                for d in gds:
                    d.wait()

                def scale_one(e, _):
                    rows_v[e, pl.ds(0, 16)] = (rows_v[e, pl.ds(0, 16)]
                                               * wex_v[e, pl.ds(0, 16)])
                    return 0
                lax.fori_loop(0, B, scale_one, 0, unroll=8)

                for j in range(16):
                    pltpu.async_copy(rows_v.at[pl.ds(j * 128, 128)],
                                     acc.at[col_v.at[j]], sem_s, add=True)
                return 0

            lax.fori_loop(0, NBATCH, batch, 0)
            drain_scat()
            plsc.subcore_barrier()

            def wo(tq, _):
                off = sid * 3128 + tq * 184
                pltpu.sync_copy(acc.at[pl.ds(off, 184)], tbuf)
                pltpu.sync_copy(tbuf, out_ref.at[pl.ds(off, 184)])
                return 0
            lax.fori_loop(0, 17, wo, 0)
            plsc.subcore_barrier()

        for k in range(nchunks):
            owner = k // cpc

            @pl.when(cid == owner)
            def _proc(k=k, out_ref=outs[k]):
                chunk_body(k, out_ref)

    return pl.kernel(body, out_type=out_type, mesh=mesh,
                     scratch_types=scratch,
                     compiler_params=pltpu.CompilerParams(
                         use_tc_tiling_on_sc=False))


@functools.cache
def _make_edge_gather():
    mesh = plsc.VectorSubcoreMesh(core_axis_name="c", subcore_axis_name="s")
    scratch = [
        pltpu.VMEM((8, 128), _i32),
        pltpu.VMEM((1024, 16), _f32),
    ]
    out_type = [jax.ShapeDtypeStruct((EPAD, 16), _f32) for _ in range(2)]
    per_w = EPAD // 32
    nb = per_w // 1024

    def body(h12, row2d, col2d, er, ec, idx_v, buf):
        cid = lax.axis_index("c")
        sid = lax.axis_index("s")
        wid = sid * 2 + cid

        def batch(bi, _):
            r0 = wid * (per_w // 128) + bi * 8
            eoff = wid * per_w + bi * 1024
            for src2d, dst in ((row2d, er), (col2d, ec)):
                pltpu.sync_copy(src2d.at[pl.ds(r0, 8)], idx_v)

                def g1(j, _):
                    pltpu.sync_copy(h12.at[idx_v.at[j]],
                                    buf.at[pl.ds(j * 128, 128)])
                    return 0
                lax.fori_loop(0, 8, g1, 0)
                pltpu.sync_copy(buf, dst.at[pl.ds(eoff, 1024)])
            return 0
        lax.fori_loop(0, nb, batch, 0)

    return pl.kernel(body, out_type=out_type, mesh=mesh,
                     scratch_types=scratch,
                     compiler_params=pltpu.CompilerParams(
                         use_tc_tiling_on_sc=False))


def _tc_stats(arrs, br):
    r = arrs[0].shape[0]
    ktot = sum(a.shape[1] for a in arrs)
    grid = r // br

    def kern(*refs):
        ins, out = refs[:-1], refs[-1]
        x = jnp.concatenate([rf[...] for rf in ins], axis=1)
        blk = jnp.stack([jnp.sum(x, 0), jnp.sum(x * x, 0)])

        @pl.when(pl.program_id(0) == 0)
        def _():
            out[...] = jnp.zeros_like(out)
        out[...] += blk

    return pl.pallas_call(
        kern, grid=(grid,),
        in_specs=[pl.BlockSpec((br, a.shape[1]), lambda i: (i, 0))
                  for a in arrs],
        out_specs=pl.BlockSpec((2, ktot), lambda i: (0, 0)),
        out_shape=jax.ShapeDtypeStruct((2, ktot), _f32))(*arrs)


def _tc_mm(arrs, s, t, w, relu, br=544):
    r = arrs[0].shape[0]
    ktot = sum(a.shape[1] for a in arrs)
    fout = w.shape[1]
    grid = r // br
    have_aff = s is not None
    extra = [s, t] if have_aff else []

    def kern(*refs):
        ins = refs[:len(arrs)]
        pos = len(arrs)
        if have_aff:
            s_ref, t_ref = refs[pos], refs[pos + 1]
            pos += 2
        w_ref, out = refs[pos], refs[pos + 1]
        x = jnp.concatenate([rf[...] for rf in ins], axis=1)
        if have_aff:
            x = x * s_ref[...] + t_ref[...]
        if relu:
            x = jnp.maximum(x, 0.0)
        out[...] = jnp.dot(x, w_ref[...], preferred_element_type=_f32)

    in_specs = [pl.BlockSpec((br, a.shape[1]), lambda i: (i, 0))
                for a in arrs]
    if have_aff:
        in_specs += [pl.BlockSpec((1, ktot), lambda i: (0, 0))] * 2
    in_specs += [pl.BlockSpec((ktot, fout), lambda i: (0, 0))]
    return pl.pallas_call(
        kern, grid=(grid,),
        in_specs=in_specs,
        out_specs=pl.BlockSpec((br, fout), lambda i: (i, 0)),
        out_shape=jax.ShapeDtypeStruct((r, fout), _f32))(*arrs, *extra, w)


def _tc_affine16(a, s, t, br=544):
    r = a.shape[0]
    grid = r // br

    def kern(a_ref, s_ref, t_ref, out):
        out[...] = a_ref[...] * s_ref[...] + t_ref[...]

    return pl.pallas_call(
        kern, grid=(grid,),
        in_specs=[pl.BlockSpec((br, 16), lambda i: (i, 0)),
                  pl.BlockSpec((1, 16), lambda i: (0, 0)),
                  pl.BlockSpec((1, 16), lambda i: (0, 0))],
        out_specs=pl.BlockSpec((br, 16), lambda i: (i, 0)),
        out_shape=jax.ShapeDtypeStruct((r, 16), _f32))(a, s, t)


def _tc_mlp_stats(er, ec, wa, wb, be=1000):
    grid = E // be

    def kern(er_ref, ec_ref, wa_ref, wb_ref, out):
        e_r, e_c = er_ref[...], ec_ref[...]
        wa_, wb_ = wa_ref[...], wb_ref[...]
        u1 = (jnp.dot(e_r, wa_, preferred_element_type=_f32)
              + jnp.dot(e_c, wb_, preferred_element_type=_f32))
        u2 = (jnp.dot(e_c, wa_, preferred_element_type=_f32)
              + jnp.dot(e_r, wb_, preferred_element_type=_f32))
        blk = jnp.stack([jnp.sum(u1, 0), jnp.sum(u1 * u1, 0),
                         jnp.sum(u2, 0), jnp.sum(u2 * u2, 0)])

        @pl.when(pl.program_id(0) == 0)
        def _():
            out[...] = jnp.zeros_like(out)
        out[...] += blk

    return pl.pallas_call(
        kern, grid=(grid,),
        in_specs=[pl.BlockSpec((be, 16), lambda i: (i, 0)),
                  pl.BlockSpec((be, 16), lambda i: (i, 0)),
                  pl.BlockSpec((16, 128), lambda i: (0, 0)),
                  pl.BlockSpec((16, 128), lambda i: (0, 0))],
        out_specs=pl.BlockSpec((4, 128), lambda i: (0, 0)),
        out_shape=jax.ShapeDtypeStruct((4, 128), _f32))(er, ec, wa, wb)


def _tc_mlp_final(er, ec, wa, wb, s1, t1, s2, t2, w2, b2, be=8000):
    grid = E // be

    def kern(er_ref, ec_ref, wa_ref, wb_ref, s1r, t1r, s2r, t2r, w2r, b2r,
             out):
        e_r, e_c = er_ref[...], ec_ref[...]
        wa_, wb_ = wa_ref[...], wb_ref[...]
        u1 = (jnp.dot(e_r, wa_, preferred_element_type=_f32)
              + jnp.dot(e_c, wb_, preferred_element_type=_f32))
        u2 = (jnp.dot(e_c, wa_, preferred_element_type=_f32)
              + jnp.dot(e_r, wb_, preferred_element_type=_f32))
        h1 = jnp.maximum(u1 * s1r[...] + t1r[...], 0.0)
        h2 = jnp.maximum(u2 * s2r[...] + t2r[...], 0.0)
        p1 = jnp.sum(h1 * w2r[...], axis=1)
        p2 = jnp.sum(h2 * w2r[...], axis=1)
        z = 0.5 * (p1 + p2) + b2r[0, 0]
        out[...] = (1.0 / (1.0 + jnp.exp(-z))).reshape(out.shape)

    cst = lambda shape: pl.BlockSpec(shape, lambda i: (0, 0))
    return pl.pallas_call(
        kern, grid=(grid,),
        in_specs=[pl.BlockSpec((be, 16), lambda i: (i, 0)),
                  pl.BlockSpec((be, 16), lambda i: (i, 0)),
                  cst((16, 128)), cst((16, 128)),
                  cst((1, 128)), cst((1, 128)), cst((1, 128)), cst((1, 128)),
                  cst((1, 128)), cst((1, 1))],
        out_specs=pl.BlockSpec((be // 1000, 1000), lambda i: (i, 0)),
        out_shape=jax.ShapeDtypeStruct((E // 1000, 1000), _f32))(
            er, ec, wa, wb, s1, t1, s2, t2, w2, b2)


def kernel(x, edge_index, edge_weight, params):
    row = edge_index[0].astype(_i32)
    col = edge_index[1].astype(_i32)
    w = edge_weight.astype(_f32)
    pad = EPAD - E
    row2d = jnp.concatenate([row, jnp.zeros((pad,), _i32)]).reshape(ROWS2D, 128)
    col2d = jnp.concatenate([col, jnp.zeros((pad,), _i32)]).reshape(ROWS2D, 128)
    w1d = jnp.concatenate([w, jnp.zeros((pad,), _f32)])
    wexp = jnp.broadcast_to(w1d[:, None], (EPAD, 16))
    zeros16 = jnp.zeros((NPAD, 16), _f32)
    convs = params['convs']
    agg1 = _make_agg(1, 1)
    agg8 = _make_agg(8, 8)

    st_x = _tc_stats([x], 1000)
    m0 = st_x[0] / N
    v0 = st_x[1] / N - m0 * m0
    s0 = params['bn0_g'] * lax.rsqrt(v0 + EPS)
    t0 = params['bn0_b'] - m0 * s0
    w0 = convs[0]['W']
    weff = (jnp.zeros((16, 128), _f32)
            .at[0].set(s0[0] * w0[0])
            .at[1].set(s0[1] * w0[1])
            .at[2].set(t0[0] * w0[0] + t0[1] * w0[1]))

    tbl0 = (jnp.zeros((NPAD, 16), _f32)
            .at[:N, 0:2].set(x)
            .at[:N, 2].set(1.0))
    g = agg1(tbl0, row2d, col2d, wexp, zeros16)[0]
    cur = [_tc_mm([g], None, None, weff, False)]

    a11 = None
    for i in range(1, 12):
        st = _tc_stats(cur, 544)
        m = st[0] / N
        v = st[1] / N - m * m
        s = convs[i - 1]['bn_g'] * lax.rsqrt(v + EPS)
        t = convs[i - 1]['bn_b'] - m * s
        if i < 11:
            hw = _tc_mm(cur, s[None], t[None], convs[i]['W'], True)
            cur = list(agg8(hw.reshape(NPAD * 8, 16), row2d, col2d, wexp,
                            zeros16))
        else:
            hw11 = _tc_mm(cur, s[None], t[None], convs[11]['W'], True)
            a11 = agg1(hw11, row2d, col2d, wexp, zeros16)[0]

    st11 = _tc_stats([a11], 544)
    m = st11[0] / N
    v = st11[1] / N - m * m
    s = convs[11]['bn_g'] * lax.rsqrt(v + EPS)
    t = convs[11]['bn_b'] - m * s
    h12 = _tc_affine16(a11, s[None], t[None])

    e_r, e_c = _make_edge_gather()(h12, row2d, col2d)

    w1 = params['mlp_W1']
    w1a, w1b = w1[:16], w1[16:]
    stm = _tc_mlp_stats(e_r, e_c, w1a, w1b)
    m1 = stm[0] / E
    v1 = stm[1] / E - m1 * m1
    m2 = stm[2] / E
    v2 = stm[3] / E - m2 * m2
    gm, bm = params['mlp_bn_g'], params['mlp_bn_b']
    s1 = gm * lax.rsqrt(v1 + EPS)
    t1 = bm - m1 * s1
    s2 = gm * lax.rsqrt(v2 + EPS)
    t2 = bm - m2 * s2
    w2row = params['mlp_W2'][:, 0][None]
    b2v = params['mlp_b2'].reshape(1, 1)
    out2 = _tc_mlp_final(e_r, e_c, w1a, w1b, s1[None], t1[None], s2[None],
                         t2[None], w2row, b2v)
    return out2.reshape(E)


# speedup vs baseline: 1.7334x; 1.0880x over previous
import functools

import jax
import jax.numpy as jnp
from jax import lax
from jax.experimental import pallas as pl
from jax.experimental.pallas import tpu as pltpu
from jax.experimental.pallas import tpu_sc as plsc

N = 50000
E = 800000
NPAD = 50048
B = 1024
NBATCH = 50
EPAD = 16 * B * NBATCH
ROWS2D = EPAD // 128
EPS = 1e-5
_f32 = jnp.float32
_i32 = jnp.int32


@functools.cache
def _make_agg(cmul: int, nchunks: int):
    cpc = max(nchunks // 2, 1)
    mesh = plsc.VectorSubcoreMesh(core_axis_name="c", subcore_axis_name="s")
    nbuf = lambda: [pltpu.VMEM((8, 128), _i32),
                    pltpu.VMEM((8, 128), _i32),
                    pltpu.VMEM((B, 16), _f32),
                    pltpu.VMEM((B, 16), _f32)]
    scratch = nbuf() + nbuf() + [
        pltpu.VMEM((184, 16), _f32),
        pltpu.VMEM_SHARED((NPAD, 16), _f32),
        pltpu.SemaphoreType.DMA,
        pltpu.SemaphoreType.DMA,
        pltpu.SemaphoreType.DMA,
    ]
    out_type = [jax.ShapeDtypeStruct((NPAD, 16), _f32) for _ in range(nchunks)]

    def body(table, rowidx, col2d, wexp, zeros, *rest):
        outs = rest[:nchunks]
        bufs = (rest[nchunks:nchunks + 4], rest[nchunks + 4:nchunks + 8])
        tbuf, acc, sem_i, sem_g, sem_s = rest[nchunks + 8:]
        cid = lax.axis_index("c")
        sid = lax.axis_index("s")

        def fire_idx(bi, bb, base_k):
            row_v, col_v, wex_v, _ = bb
            r0 = sid * (NBATCH * 8) + bi * 8
            e0 = sid * (NBATCH * B) + bi * B
            pltpu.async_copy(rowidx.at[pl.ds(base_k + r0, 8)], row_v, sem_i)
            pltpu.async_copy(col2d.at[pl.ds(r0, 8)], col_v, sem_i)
            pltpu.async_copy(wexp.at[pl.ds(e0, B)], wex_v, sem_i)

        def wait_idx(bb):
            row_v, col_v, wex_v, _ = bb
            pltpu.make_async_copy(rowidx.at[pl.ds(0, 8)], row_v, sem_i).wait()
            pltpu.make_async_copy(col2d.at[pl.ds(0, 8)], col_v, sem_i).wait()
            pltpu.make_async_copy(wexp.at[pl.ds(0, B)], wex_v, sem_i).wait()

        def fire_gather(bb):
            row_v, _, _, rows_v = bb
            for j in range(8):
                pltpu.async_copy(table.at[row_v.at[j]],
                                 rows_v.at[pl.ds(j * 128, 128)], sem_g)

        def wait_gather(bb):
            row_v, _, _, rows_v = bb
            for j in range(8):
                pltpu.make_async_copy(table.at[row_v.at[j]],
                                      rows_v.at[pl.ds(j * 128, 128)],
                                      sem_g).wait()

        def scale(bb):
            _, _, wex_v, rows_v = bb

            def scale_one(e, _):
                rows_v[e, pl.ds(0, 16)] = (rows_v[e, pl.ds(0, 16)]
                                           * wex_v[e, pl.ds(0, 16)])
                return 0
            lax.fori_loop(0, B, scale_one, 0, unroll=8)

        def fire_scat(bb):
            _, col_v, _, rows_v = bb
            for j in range(8):
                pltpu.async_copy(rows_v.at[pl.ds(j * 128, 128)],
                                 acc.at[col_v.at[j]], sem_s, add=True)

        def drain_scat(bb):
            _, col_v, _, rows_v = bb
            for j in range(8):
                pltpu.make_async_copy(rows_v.at[pl.ds(j * 128, 128)],
                                      acc.at[col_v.at[j]], sem_s).wait()

        def chunk_body(k, out_ref):
            base_k = k * ROWS2D if cmul != 1 else 0

            @pl.when(sid == 0)
            def _zero():
                pltpu.sync_copy(zeros, acc)
            plsc.subcore_barrier()

            fire_idx(0, bufs[0], base_k)
            wait_idx(bufs[0])
            fire_gather(bufs[0])

            def step(si, _):
                for par in (0, 1):
                    bi = si * 2 + par
                    cur, nxt = bufs[par], bufs[1 - par]
                    wait_gather(cur)

                    @pl.when(bi + 1 < NBATCH)
                    def _pf(bi=bi, nxt=nxt):
                        fire_idx(bi + 1, nxt, base_k)
                    scale(cur)

                    @pl.when(bi + 1 < NBATCH)
                    def _pf2(bi=bi, nxt=nxt):
                        wait_idx(nxt)

                        @pl.when(bi > 0)
                        def _dr(nxt=nxt):
                            drain_scat(nxt)
                        fire_gather(nxt)
                    fire_scat(cur)
                return 0

            lax.fori_loop(0, NBATCH // 2, step, 0)
            drain_scat(bufs[0])
            drain_scat(bufs[1])
            plsc.subcore_barrier()

            def wo(tq, _):
                off = sid * 3128 + tq * 184
                pltpu.sync_copy(acc.at[pl.ds(off, 184)], tbuf)
                pltpu.sync_copy(tbuf, out_ref.at[pl.ds(off, 184)])
                return 0
            lax.fori_loop(0, 17, wo, 0)
            plsc.subcore_barrier()

        for k in range(nchunks):
            owner = k // cpc

            @pl.when(cid == owner)
            def _proc(k=k, out_ref=outs[k]):
                chunk_body(k, out_ref)

    return pl.kernel(body, out_type=out_type, mesh=mesh,
                     scratch_types=scratch,
                     compiler_params=pltpu.CompilerParams(
                         use_tc_tiling_on_sc=False))


@functools.cache
def _make_edge_gather():
    mesh = plsc.VectorSubcoreMesh(core_axis_name="c", subcore_axis_name="s")
    scratch = [
        pltpu.VMEM((8, 128), _i32),
        pltpu.VMEM((1024, 16), _f32),
    ]
    out_type = [jax.ShapeDtypeStruct((EPAD, 16), _f32) for _ in range(2)]
    per_w = EPAD // 32
    nb = per_w // 1024

    def body(h12, row2d, col2d, er, ec, idx_v, buf):
        cid = lax.axis_index("c")
        sid = lax.axis_index("s")
        wid = sid * 2 + cid

        def batch(bi, _):
            r0 = wid * (per_w // 128) + bi * 8
            eoff = wid * per_w + bi * 1024
            for src2d, dst in ((row2d, er), (col2d, ec)):
                pltpu.sync_copy(src2d.at[pl.ds(r0, 8)], idx_v)

                def g1(j, _):
                    pltpu.sync_copy(h12.at[idx_v.at[j]],
                                    buf.at[pl.ds(j * 128, 128)])
                    return 0
                lax.fori_loop(0, 8, g1, 0)
                pltpu.sync_copy(buf, dst.at[pl.ds(eoff, 1024)])
            return 0
        lax.fori_loop(0, nb, batch, 0)

    return pl.kernel(body, out_type=out_type, mesh=mesh,
                     scratch_types=scratch,
                     compiler_params=pltpu.CompilerParams(
                         use_tc_tiling_on_sc=False))


def _tc_stats(arrs, br):
    r = arrs[0].shape[0]
    ktot = sum(a.shape[1] for a in arrs)
    grid = r // br

    def kern(*refs):
        ins, out = refs[:-1], refs[-1]
        x = jnp.concatenate([rf[...] for rf in ins], axis=1)
        blk = jnp.stack([jnp.sum(x, 0), jnp.sum(x * x, 0)])

        @pl.when(pl.program_id(0) == 0)
        def _():
            out[...] = jnp.zeros_like(out)
        out[...] += blk

    return pl.pallas_call(
        kern, grid=(grid,),
        in_specs=[pl.BlockSpec((br, a.shape[1]), lambda i: (i, 0))
                  for a in arrs],
        out_specs=pl.BlockSpec((2, ktot), lambda i: (0, 0)),
        out_shape=jax.ShapeDtypeStruct((2, ktot), _f32))(*arrs)


def _tc_mm(arrs, s, t, w, relu, br=544):
    r = arrs[0].shape[0]
    ktot = sum(a.shape[1] for a in arrs)
    fout = w.shape[1]
    grid = r // br
    have_aff = s is not None
    extra = [s, t] if have_aff else []

    def kern(*refs):
        ins = refs[:len(arrs)]
        pos = len(arrs)
        if have_aff:
            s_ref, t_ref = refs[pos], refs[pos + 1]
            pos += 2
        w_ref, out = refs[pos], refs[pos + 1]
        x = jnp.concatenate([rf[...] for rf in ins], axis=1)
        if have_aff:
            x = x * s_ref[...] + t_ref[...]
        if relu:
            x = jnp.maximum(x, 0.0)
        out[...] = jnp.dot(x, w_ref[...], preferred_element_type=_f32)

    in_specs = [pl.BlockSpec((br, a.shape[1]), lambda i: (i, 0))
                for a in arrs]
    if have_aff:
        in_specs += [pl.BlockSpec((1, ktot), lambda i: (0, 0))] * 2
    in_specs += [pl.BlockSpec((ktot, fout), lambda i: (0, 0))]
    return pl.pallas_call(
        kern, grid=(grid,),
        in_specs=in_specs,
        out_specs=pl.BlockSpec((br, fout), lambda i: (i, 0)),
        out_shape=jax.ShapeDtypeStruct((r, fout), _f32))(*arrs, *extra, w)


def _tc_affine16(a, s, t, br=544):
    r = a.shape[0]
    grid = r // br

    def kern(a_ref, s_ref, t_ref, out):
        out[...] = a_ref[...] * s_ref[...] + t_ref[...]

    return pl.pallas_call(
        kern, grid=(grid,),
        in_specs=[pl.BlockSpec((br, 16), lambda i: (i, 0)),
                  pl.BlockSpec((1, 16), lambda i: (0, 0)),
                  pl.BlockSpec((1, 16), lambda i: (0, 0))],
        out_specs=pl.BlockSpec((br, 16), lambda i: (i, 0)),
        out_shape=jax.ShapeDtypeStruct((r, 16), _f32))(a, s, t)


def _tc_mlp_stats(er, ec, wa, wb, be=1000):
    grid = E // be

    def kern(er_ref, ec_ref, wa_ref, wb_ref, out):
        e_r, e_c = er_ref[...], ec_ref[...]
        wa_, wb_ = wa_ref[...], wb_ref[...]
        u1 = (jnp.dot(e_r, wa_, preferred_element_type=_f32)
              + jnp.dot(e_c, wb_, preferred_element_type=_f32))
        u2 = (jnp.dot(e_c, wa_, preferred_element_type=_f32)
              + jnp.dot(e_r, wb_, preferred_element_type=_f32))
        blk = jnp.stack([jnp.sum(u1, 0), jnp.sum(u1 * u1, 0),
                         jnp.sum(u2, 0), jnp.sum(u2 * u2, 0)])

        @pl.when(pl.program_id(0) == 0)
        def _():
            out[...] = jnp.zeros_like(out)
        out[...] += blk

    return pl.pallas_call(
        kern, grid=(grid,),
        in_specs=[pl.BlockSpec((be, 16), lambda i: (i, 0)),
                  pl.BlockSpec((be, 16), lambda i: (i, 0)),
                  pl.BlockSpec((16, 128), lambda i: (0, 0)),
                  pl.BlockSpec((16, 128), lambda i: (0, 0))],
        out_specs=pl.BlockSpec((4, 128), lambda i: (0, 0)),
        out_shape=jax.ShapeDtypeStruct((4, 128), _f32))(er, ec, wa, wb)


def _tc_mlp_final(er, ec, wa, wb, s1, t1, s2, t2, w2, b2, be=8000):
    grid = E // be

    def kern(er_ref, ec_ref, wa_ref, wb_ref, s1r, t1r, s2r, t2r, w2r, b2r,
             out):
        e_r, e_c = er_ref[...], ec_ref[...]
        wa_, wb_ = wa_ref[...], wb_ref[...]
        u1 = (jnp.dot(e_r, wa_, preferred_element_type=_f32)
              + jnp.dot(e_c, wb_, preferred_element_type=_f32))
        u2 = (jnp.dot(e_c, wa_, preferred_element_type=_f32)
              + jnp.dot(e_r, wb_, preferred_element_type=_f32))
        h1 = jnp.maximum(u1 * s1r[...] + t1r[...], 0.0)
        h2 = jnp.maximum(u2 * s2r[...] + t2r[...], 0.0)
        p1 = jnp.sum(h1 * w2r[...], axis=1)
        p2 = jnp.sum(h2 * w2r[...], axis=1)
        z = 0.5 * (p1 + p2) + b2r[0, 0]
        out[...] = (1.0 / (1.0 + jnp.exp(-z))).reshape(out.shape)

    cst = lambda shape: pl.BlockSpec(shape, lambda i: (0, 0))
    return pl.pallas_call(
        kern, grid=(grid,),
        in_specs=[pl.BlockSpec((be, 16), lambda i: (i, 0)),
                  pl.BlockSpec((be, 16), lambda i: (i, 0)),
                  cst((16, 128)), cst((16, 128)),
                  cst((1, 128)), cst((1, 128)), cst((1, 128)), cst((1, 128)),
                  cst((1, 128)), cst((1, 1))],
        out_specs=pl.BlockSpec((be // 1000, 1000), lambda i: (i, 0)),
        out_shape=jax.ShapeDtypeStruct((E // 1000, 1000), _f32))(
            er, ec, wa, wb, s1, t1, s2, t2, w2, b2)


def kernel(x, edge_index, edge_weight, params):
    row = edge_index[0].astype(_i32)
    col = edge_index[1].astype(_i32)
    w = edge_weight.astype(_f32)
    pad = EPAD - E
    row2d = jnp.concatenate([row, jnp.zeros((pad,), _i32)]).reshape(ROWS2D, 128)
    col2d = jnp.concatenate([col, jnp.zeros((pad,), _i32)]).reshape(ROWS2D, 128)
    w1d = jnp.concatenate([w, jnp.zeros((pad,), _f32)])
    wexp = jnp.broadcast_to(w1d[:, None], (EPAD, 16))
    rowk8 = (row2d[None, :, :] * 8
             + jnp.arange(8, dtype=_i32)[:, None, None]).reshape(
                 8 * ROWS2D, 128)
    zeros16 = jnp.zeros((NPAD, 16), _f32)
    convs = params['convs']
    agg1 = _make_agg(1, 1)
    agg8 = _make_agg(8, 8)

    st_x = _tc_stats([x], 1000)
    m0 = st_x[0] / N
    v0 = st_x[1] / N - m0 * m0
    s0 = params['bn0_g'] * lax.rsqrt(v0 + EPS)
    t0 = params['bn0_b'] - m0 * s0
    w0 = convs[0]['W']
    weff = (jnp.zeros((16, 128), _f32)
            .at[0].set(s0[0] * w0[0])
            .at[1].set(s0[1] * w0[1])
            .at[2].set(t0[0] * w0[0] + t0[1] * w0[1]))

    tbl0 = (jnp.zeros((NPAD, 16), _f32)
            .at[:N, 0:2].set(x)
            .at[:N, 2].set(1.0))
    g = agg1(tbl0, row2d, col2d, wexp, zeros16)[0]
    cur = [_tc_mm([g], None, None, weff, False)]

    a11 = None
    for i in range(1, 12):
        st = _tc_stats(cur, 544)
        m = st[0] / N
        v = st[1] / N - m * m
        s = convs[i - 1]['bn_g'] * lax.rsqrt(v + EPS)
        t = convs[i - 1]['bn_b'] - m * s
        if i < 11:
            hw = _tc_mm(cur, s[None], t[None], convs[i]['W'], True)
            cur = list(agg8(hw.reshape(NPAD * 8, 16), rowk8, col2d, wexp,
                            zeros16))
        else:
            hw11 = _tc_mm(cur, s[None], t[None], convs[11]['W'], True)
            a11 = agg1(hw11, row2d, col2d, wexp, zeros16)[0]

    st11 = _tc_stats([a11], 544)
    m = st11[0] / N
    v = st11[1] / N - m * m
    s = convs[11]['bn_g'] * lax.rsqrt(v + EPS)
    t = convs[11]['bn_b'] - m * s
    h12 = _tc_affine16(a11, s[None], t[None])

    e_r, e_c = _make_edge_gather()(h12, row2d, col2d)

    w1 = params['mlp_W1']
    w1a, w1b = w1[:16], w1[16:]
    stm = _tc_mlp_stats(e_r, e_c, w1a, w1b)
    m1 = stm[0] / E
    v1 = stm[1] / E - m1 * m1
    m2 = stm[2] / E
    v2 = stm[3] / E - m2 * m2
    gm, bm = params['mlp_bn_g'], params['mlp_bn_b']
    s1 = gm * lax.rsqrt(v1 + EPS)
    t1 = bm - m1 * s1
    s2 = gm * lax.rsqrt(v2 + EPS)
    t2 = bm - m2 * s2
    w2row = params['mlp_W2'][:, 0][None]
    b2v = params['mlp_b2'].reshape(1, 1)
    out2 = _tc_mlp_final(e_r, e_c, w1a, w1b, s1[None], t1[None], s2[None],
                         t2[None], w2row, b2v)
    return out2.reshape(E)


# speedup vs baseline: 1.8713x; 1.0795x over previous
import functools

import jax
import jax.numpy as jnp
from jax import lax
from jax.experimental import pallas as pl
from jax.experimental.pallas import tpu as pltpu
from jax.experimental.pallas import tpu_sc as plsc

N = 50000
E = 800000
NPAD = 50048
B = 1024
NBATCH = 50
EPAD = 16 * B * NBATCH
ROWS2D = EPAD // 128
EPS = 1e-5
_f32 = jnp.float32
_i32 = jnp.int32


@functools.cache
def _make_agg(cmul: int, nchunks: int):
    cpc = max(nchunks // 2, 1)
    mesh = plsc.VectorSubcoreMesh(core_axis_name="c", subcore_axis_name="s")
    nbuf = lambda: [pltpu.VMEM((8, 128), _i32),
                    pltpu.VMEM((8, 128), _i32),
                    pltpu.VMEM((B, 16), _f32),
                    pltpu.VMEM((B, 16), _f32)]
    scratch = nbuf() + nbuf() + [
        pltpu.VMEM((184, 16), _f32),
        pltpu.VMEM_SHARED((NPAD, 16), _f32),
        pltpu.SemaphoreType.DMA,
        pltpu.SemaphoreType.DMA,
        pltpu.SemaphoreType.DMA,
    ]
    out_type = [jax.ShapeDtypeStruct((NPAD, 16), _f32) for _ in range(nchunks)]

    def body(table, rowidx, col2d, wexp, zeros, *rest):
        outs = rest[:nchunks]
        bufs = (rest[nchunks:nchunks + 4], rest[nchunks + 4:nchunks + 8])
        tbuf, acc, sem_i, sem_g, sem_s = rest[nchunks + 8:]
        cid = lax.axis_index("c")
        sid = lax.axis_index("s")

        def fire_idx(bi, bb, base_k):
            row_v, col_v, wex_v, _ = bb
            r0 = sid * (NBATCH * 8) + bi * 8
            e0 = sid * (NBATCH * B) + bi * B
            pltpu.async_copy(rowidx.at[pl.ds(base_k + r0, 8)], row_v, sem_i)
            pltpu.async_copy(col2d.at[pl.ds(r0, 8)], col_v, sem_i)
            pltpu.async_copy(wexp.at[pl.ds(e0, B)], wex_v, sem_i)

        def wait_idx(bb):
            row_v, col_v, wex_v, _ = bb
            pltpu.make_async_copy(rowidx.at[pl.ds(0, 8)], row_v, sem_i).wait()
            pltpu.make_async_copy(col2d.at[pl.ds(0, 8)], col_v, sem_i).wait()
            pltpu.make_async_copy(wexp.at[pl.ds(0, B)], wex_v, sem_i).wait()

        def fire_gather(bb):
            row_v, _, _, rows_v = bb
            for j in range(8):
                pltpu.async_copy(table.at[row_v.at[j]],
                                 rows_v.at[pl.ds(j * 128, 128)], sem_g)

        def wait_gather(bb):
            row_v, _, _, rows_v = bb
            for j in range(8):
                pltpu.make_async_copy(table.at[row_v.at[j]],
                                      rows_v.at[pl.ds(j * 128, 128)],
                                      sem_g).wait()

        def scale(bb):
            _, _, wex_v, rows_v = bb

            def scale_one(e, _):
                rows_v[e, pl.ds(0, 16)] = (rows_v[e, pl.ds(0, 16)]
                                           * wex_v[e, pl.ds(0, 16)])
                return 0
            lax.fori_loop(0, B, scale_one, 0, unroll=8)

        def fire_scat(bb):
            _, col_v, _, rows_v = bb
            for j in range(8):
                pltpu.async_copy(rows_v.at[pl.ds(j * 128, 128)],
                                 acc.at[col_v.at[j]], sem_s, add=True)

        def drain_scat(bb):
            _, col_v, _, rows_v = bb
            for j in range(8):
                pltpu.make_async_copy(rows_v.at[pl.ds(j * 128, 128)],
                                      acc.at[col_v.at[j]], sem_s).wait()

        def chunk_body(k, out_ref):
            base_k = k * ROWS2D if cmul != 1 else 0

            @pl.when(sid == 0)
            def _zero():
                pltpu.sync_copy(zeros, acc)
            plsc.subcore_barrier()

            fire_idx(0, bufs[0], base_k)
            wait_idx(bufs[0])
            fire_gather(bufs[0])

            def step(si, _):
                for par in (0, 1):
                    bi = si * 2 + par
                    cur, nxt = bufs[par], bufs[1 - par]
                    wait_gather(cur)

                    @pl.when(bi + 1 < NBATCH)
                    def _pf(bi=bi, nxt=nxt):
                        fire_idx(bi + 1, nxt, base_k)
                        wait_idx(nxt)

                        @pl.when(bi > 0)
                        def _dr(nxt=nxt):
                            drain_scat(nxt)
                        fire_gather(nxt)
                    scale(cur)
                    fire_scat(cur)
                return 0

            lax.fori_loop(0, NBATCH // 2, step, 0)
            drain_scat(bufs[0])
            drain_scat(bufs[1])
            plsc.subcore_barrier()

            def wo(tq, _):
                off = sid * 3128 + tq * 184
                pltpu.sync_copy(acc.at[pl.ds(off, 184)], tbuf)
                pltpu.sync_copy(tbuf, out_ref.at[pl.ds(off, 184)])
                return 0
            lax.fori_loop(0, 17, wo, 0)
            plsc.subcore_barrier()

        for k in range(nchunks):
            owner = k // cpc

            @pl.when(cid == owner)
            def _proc(k=k, out_ref=outs[k]):
                chunk_body(k, out_ref)

    return pl.kernel(body, out_type=out_type, mesh=mesh,
                     scratch_types=scratch,
                     compiler_params=pltpu.CompilerParams(
                         use_tc_tiling_on_sc=False))


@functools.cache
def _make_edge_gather():
    mesh = plsc.VectorSubcoreMesh(core_axis_name="c", subcore_axis_name="s")
    scratch = [
        pltpu.VMEM((8, 128), _i32),
        pltpu.VMEM((1024, 16), _f32),
    ]
    out_type = [jax.ShapeDtypeStruct((EPAD, 16), _f32) for _ in range(2)]
    per_w = EPAD // 32
    nb = per_w // 1024

    def body(h12, row2d, col2d, er, ec, idx_v, buf):
        cid = lax.axis_index("c")
        sid = lax.axis_index("s")
        wid = sid * 2 + cid

        def batch(bi, _):
            r0 = wid * (per_w // 128) + bi * 8
            eoff = wid * per_w + bi * 1024
            for src2d, dst in ((row2d, er), (col2d, ec)):
                pltpu.sync_copy(src2d.at[pl.ds(r0, 8)], idx_v)

                def g1(j, _):
                    pltpu.sync_copy(h12.at[idx_v.at[j]],
                                    buf.at[pl.ds(j * 128, 128)])
                    return 0
                lax.fori_loop(0, 8, g1, 0)
                pltpu.sync_copy(buf, dst.at[pl.ds(eoff, 1024)])
            return 0
        lax.fori_loop(0, nb, batch, 0)

    return pl.kernel(body, out_type=out_type, mesh=mesh,
                     scratch_types=scratch,
                     compiler_params=pltpu.CompilerParams(
                         use_tc_tiling_on_sc=False))


def _tc_stats(arrs, br):
    r = arrs[0].shape[0]
    ktot = sum(a.shape[1] for a in arrs)
    grid = r // br

    def kern(*refs):
        ins, out = refs[:-1], refs[-1]
        x = jnp.concatenate([rf[...] for rf in ins], axis=1)
        blk = jnp.stack([jnp.sum(x, 0), jnp.sum(x * x, 0)])

        @pl.when(pl.program_id(0) == 0)
        def _():
            out[...] = jnp.zeros_like(out)
        out[...] += blk

    return pl.pallas_call(
        kern, grid=(grid,),
        in_specs=[pl.BlockSpec((br, a.shape[1]), lambda i: (i, 0))
                  for a in arrs],
        out_specs=pl.BlockSpec((2, ktot), lambda i: (0, 0)),
        out_shape=jax.ShapeDtypeStruct((2, ktot), _f32))(*arrs)


def _tc_mm(arrs, s, t, w, relu, br=544):
    r = arrs[0].shape[0]
    ktot = sum(a.shape[1] for a in arrs)
    fout = w.shape[1]
    grid = r // br
    have_aff = s is not None
    extra = [s, t] if have_aff else []

    def kern(*refs):
        ins = refs[:len(arrs)]
        pos = len(arrs)
        if have_aff:
            s_ref, t_ref = refs[pos], refs[pos + 1]
            pos += 2
        w_ref, out = refs[pos], refs[pos + 1]
        x = jnp.concatenate([rf[...] for rf in ins], axis=1)
        if have_aff:
            x = x * s_ref[...] + t_ref[...]
        if relu:
            x = jnp.maximum(x, 0.0)
        out[...] = jnp.dot(x, w_ref[...], preferred_element_type=_f32)

    in_specs = [pl.BlockSpec((br, a.shape[1]), lambda i: (i, 0))
                for a in arrs]
    if have_aff:
        in_specs += [pl.BlockSpec((1, ktot), lambda i: (0, 0))] * 2
    in_specs += [pl.BlockSpec((ktot, fout), lambda i: (0, 0))]
    return pl.pallas_call(
        kern, grid=(grid,),
        in_specs=in_specs,
        out_specs=pl.BlockSpec((br, fout), lambda i: (i, 0)),
        out_shape=jax.ShapeDtypeStruct((r, fout), _f32))(*arrs, *extra, w)


def _tc_affine16(a, s, t, br=544):
    r = a.shape[0]
    grid = r // br

    def kern(a_ref, s_ref, t_ref, out):
        out[...] = a_ref[...] * s_ref[...] + t_ref[...]

    return pl.pallas_call(
        kern, grid=(grid,),
        in_specs=[pl.BlockSpec((br, 16), lambda i: (i, 0)),
                  pl.BlockSpec((1, 16), lambda i: (0, 0)),
                  pl.BlockSpec((1, 16), lambda i: (0, 0))],
        out_specs=pl.BlockSpec((br, 16), lambda i: (i, 0)),
        out_shape=jax.ShapeDtypeStruct((r, 16), _f32))(a, s, t)


def _tc_mlp_stats(er, ec, wa, wb, be=1000):
    grid = E // be

    def kern(er_ref, ec_ref, wa_ref, wb_ref, out):
        e_r, e_c = er_ref[...], ec_ref[...]
        wa_, wb_ = wa_ref[...], wb_ref[...]
        u1 = (jnp.dot(e_r, wa_, preferred_element_type=_f32)
              + jnp.dot(e_c, wb_, preferred_element_type=_f32))
        u2 = (jnp.dot(e_c, wa_, preferred_element_type=_f32)
              + jnp.dot(e_r, wb_, preferred_element_type=_f32))
        blk = jnp.stack([jnp.sum(u1, 0), jnp.sum(u1 * u1, 0),
                         jnp.sum(u2, 0), jnp.sum(u2 * u2, 0)])

        @pl.when(pl.program_id(0) == 0)
        def _():
            out[...] = jnp.zeros_like(out)
        out[...] += blk

    return pl.pallas_call(
        kern, grid=(grid,),
        in_specs=[pl.BlockSpec((be, 16), lambda i: (i, 0)),
                  pl.BlockSpec((be, 16), lambda i: (i, 0)),
                  pl.BlockSpec((16, 128), lambda i: (0, 0)),
                  pl.BlockSpec((16, 128), lambda i: (0, 0))],
        out_specs=pl.BlockSpec((4, 128), lambda i: (0, 0)),
        out_shape=jax.ShapeDtypeStruct((4, 128), _f32))(er, ec, wa, wb)


def _tc_mlp_final(er, ec, wa, wb, s1, t1, s2, t2, w2, b2, be=8000):
    grid = E // be

    def kern(er_ref, ec_ref, wa_ref, wb_ref, s1r, t1r, s2r, t2r, w2r, b2r,
             out):
        e_r, e_c = er_ref[...], ec_ref[...]
        wa_, wb_ = wa_ref[...], wb_ref[...]
        u1 = (jnp.dot(e_r, wa_, preferred_element_type=_f32)
              + jnp.dot(e_c, wb_, preferred_element_type=_f32))
        u2 = (jnp.dot(e_c, wa_, preferred_element_type=_f32)
              + jnp.dot(e_r, wb_, preferred_element_type=_f32))
        h1 = jnp.maximum(u1 * s1r[...] + t1r[...], 0.0)
        h2 = jnp.maximum(u2 * s2r[...] + t2r[...], 0.0)
        p1 = jnp.sum(h1 * w2r[...], axis=1)
        p2 = jnp.sum(h2 * w2r[...], axis=1)
        z = 0.5 * (p1 + p2) + b2r[0, 0]
        out[...] = (1.0 / (1.0 + jnp.exp(-z))).reshape(out.shape)

    cst = lambda shape: pl.BlockSpec(shape, lambda i: (0, 0))
    return pl.pallas_call(
        kern, grid=(grid,),
        in_specs=[pl.BlockSpec((be, 16), lambda i: (i, 0)),
                  pl.BlockSpec((be, 16), lambda i: (i, 0)),
                  cst((16, 128)), cst((16, 128)),
                  cst((1, 128)), cst((1, 128)), cst((1, 128)), cst((1, 128)),
                  cst((1, 128)), cst((1, 1))],
        out_specs=pl.BlockSpec((be // 1000, 1000), lambda i: (i, 0)),
        out_shape=jax.ShapeDtypeStruct((E // 1000, 1000), _f32))(
            er, ec, wa, wb, s1, t1, s2, t2, w2, b2)


def kernel(x, edge_index, edge_weight, params):
    row = edge_index[0].astype(_i32)
    col = edge_index[1].astype(_i32)
    w = edge_weight.astype(_f32)
    pad = EPAD - E
    row2d = jnp.concatenate([row, jnp.zeros((pad,), _i32)]).reshape(ROWS2D, 128)
    col2d = jnp.concatenate([col, jnp.zeros((pad,), _i32)]).reshape(ROWS2D, 128)
    w1d = jnp.concatenate([w, jnp.zeros((pad,), _f32)])
    wexp = jnp.broadcast_to(w1d[:, None], (EPAD, 16))
    rowk8 = (row2d[None, :, :] * 8
             + jnp.arange(8, dtype=_i32)[:, None, None]).reshape(
                 8 * ROWS2D, 128)
    zeros16 = jnp.zeros((NPAD, 16), _f32)
    convs = params['convs']
    agg1 = _make_agg(1, 1)
    agg8 = _make_agg(8, 8)

    st_x = _tc_stats([x], 1000)
    m0 = st_x[0] / N
    v0 = st_x[1] / N - m0 * m0
    s0 = params['bn0_g'] * lax.rsqrt(v0 + EPS)
    t0 = params['bn0_b'] - m0 * s0
    w0 = convs[0]['W']
    weff = (jnp.zeros((16, 128), _f32)
            .at[0].set(s0[0] * w0[0])
            .at[1].set(s0[1] * w0[1])
            .at[2].set(t0[0] * w0[0] + t0[1] * w0[1]))

    tbl0 = (jnp.zeros((NPAD, 16), _f32)
            .at[:N, 0:2].set(x)
            .at[:N, 2].set(1.0))
    g = agg1(tbl0, row2d, col2d, wexp, zeros16)[0]
    cur = [_tc_mm([g], None, None, weff, False)]

    a11 = None
    for i in range(1, 12):
        st = _tc_stats(cur, 544)
        m = st[0] / N
        v = st[1] / N - m * m
        s = convs[i - 1]['bn_g'] * lax.rsqrt(v + EPS)
        t = convs[i - 1]['bn_b'] - m * s
        if i < 11:
            hw = _tc_mm(cur, s[None], t[None], convs[i]['W'], True)
            cur = list(agg8(hw.reshape(NPAD * 8, 16), rowk8, col2d, wexp,
                            zeros16))
        else:
            hw11 = _tc_mm(cur, s[None], t[None], convs[11]['W'], True)
            a11 = agg1(hw11, row2d, col2d, wexp, zeros16)[0]

    st11 = _tc_stats([a11], 544)
    m = st11[0] / N
    v = st11[1] / N - m * m
    s = convs[11]['bn_g'] * lax.rsqrt(v + EPS)
    t = convs[11]['bn_b'] - m * s
    h12 = _tc_affine16(a11, s[None], t[None])

    e_r, e_c = _make_edge_gather()(h12, row2d, col2d)

    w1 = params['mlp_W1']
    w1a, w1b = w1[:16], w1[16:]
    stm = _tc_mlp_stats(e_r, e_c, w1a, w1b)
    m1 = stm[0] / E
    v1 = stm[1] / E - m1 * m1
    m2 = stm[2] / E
    v2 = stm[3] / E - m2 * m2
    gm, bm = params['mlp_bn_g'], params['mlp_bn_b']
    s1 = gm * lax.rsqrt(v1 + EPS)
    t1 = bm - m1 * s1
    s2 = gm * lax.rsqrt(v2 + EPS)
    t2 = bm - m2 * s2
    w2row = params['mlp_W2'][:, 0][None]
    b2v = params['mlp_b2'].reshape(1, 1)
    out2 = _tc_mlp_final(e_r, e_c, w1a, w1b, s1[None], t1[None], s2[None],
                         t2[None], w2row, b2v)
    return out2.reshape(E)
